# baseline (speedup 1.0000x reference)
import numpy as np
import jax
import jax.numpy as jnp
from jax import lax
from jax.experimental import pallas as pl
from jax.experimental.pallas import tpu as pltpu

_FEAT_STRIDE = 16
_SCALES = np.array([8.0, 16.0, 32.0])
_RATIOS = np.array([0.5, 1.0, 2.0])
_PRE_NMS = 6000
_POST_NMS = 300
_NMS_THRESH = 0.7
_A = 9


def _whctrs(a):
    w = a[2] - a[0] + 1.0
    h = a[3] - a[1] + 1.0
    xc = a[0] + 0.5 * (w - 1.0)
    yc = a[1] + 0.5 * (h - 1.0)
    return w, h, xc, yc


def _mkanchors(ws, hs, xc, yc):
    ws = ws[:, None]
    hs = hs[:, None]
    return np.hstack([xc - 0.5 * (ws - 1.0), yc - 0.5 * (hs - 1.0),
                      xc + 0.5 * (ws - 1.0), yc + 0.5 * (hs - 1.0)])


def _ratio_enum(a, ratios):
    w, h, xc, yc = _whctrs(a)
    size = w * h
    size_ratios = size / ratios
    ws = np.round(np.sqrt(size_ratios))
    hs = np.round(ws * ratios)
    return _mkanchors(ws, hs, xc, yc)


def _scale_enum(a, scales):
    w, h, xc, yc = _whctrs(a)
    ws = w * scales
    hs = h * scales
    return _mkanchors(ws, hs, xc, yc)


def _gen_anchors(base_size=16):
    base = np.array([0.0, 0.0, base_size - 1.0, base_size - 1.0])
    ra = _ratio_enum(base, _RATIOS)
    return np.vstack([_scale_enum(ra[i, :], _SCALES) for i in range(ra.shape[0])])


def _all_anchor_geom(fh, fw):
    anc = _gen_anchors().astype(np.float32)
    sx = (np.arange(fw, dtype=np.float32) * np.float32(_FEAT_STRIDE))
    sy = (np.arange(fh, dtype=np.float32) * np.float32(_FEAT_STRIDE))
    SX, SY = np.meshgrid(sx, sy)
    shifts = np.stack([SX.ravel(), SY.ravel(), SX.ravel(), SY.ravel()], axis=1).astype(np.float32)
    a = (anc[None, :, :] + shifts[:, None, :]).reshape(-1, 4)
    w = (a[:, 2] - a[:, 0]) + np.float32(1.0)
    h = (a[:, 3] - a[:, 1]) + np.float32(1.0)
    cx = a[:, 0] + np.float32(0.5) * w
    cy = a[:, 1] + np.float32(0.5) * h
    return w, h, cx, cy


def _decode_body(dx, dy, dw, dh, aw, ah, acx, acy, imi, x1o, y1o, x2o, y2o):
    widths = aw[...]
    heights = ah[...]
    ctr_x = acx[...]
    ctr_y = acy[...]
    pcx = dx[...] * widths + ctr_x
    pcy = dy[...] * heights + ctr_y
    pw = jnp.exp(dw[...]) * widths
    ph = jnp.exp(dh[...]) * heights
    x1 = pcx - 0.5 * pw
    y1 = pcy - 0.5 * ph
    x2 = pcx + 0.5 * pw
    y2 = pcy + 0.5 * ph
    im_h = imi[:, 0:1]
    im_w = imi[:, 1:2]
    x1o[...] = jnp.clip(x1, 0.0, im_w - 1.0)
    x2o[...] = jnp.clip(x2, 0.0, im_w - 1.0)
    y1o[...] = jnp.clip(y1, 0.0, im_h - 1.0)
    y2o[...] = jnp.clip(y2, 0.0, im_h - 1.0)


def kernel(scores, bbox_deltas, im_info, cfg_key):
    batch = scores.shape[0]
    fh, fw = scores.shape[2], scores.shape[3]
    n = fh * fw * _A
    sc = scores[:, _A:, :, :].transpose(0, 2, 3, 1).reshape(batch, n)
    deltas = bbox_deltas.transpose(0, 2, 3, 1).reshape(batch, n, 4)
    dx, dy, dw, dh = (deltas[..., i] for i in range(4))
    aw, ah, acx, acy = (jnp.asarray(v)[None, :] for v in _all_anchor_geom(fh, fw))
    imi = jnp.pad(im_info, ((0, 0), (0, 126)))

    outs = pl.pallas_call(
        _decode_body,
        out_shape=[jax.ShapeDtypeStruct((batch, n), jnp.float32)] * 4,
    )(dx, dy, dw, dh, aw, ah, acx, acy, imi)
    x1, y1, x2, y2 = outs
    proposals = jnp.stack([x1, y1, x2, y2], axis=-1)

    def nms_single(props, s):
        top_s, order = lax.top_k(s, _PRE_NMS)
        p = props[order]
        bx1, by1, bx2, by2 = p[:, 0], p[:, 1], p[:, 2], p[:, 3]
        areas = (bx2 - bx1 + 1.0) * (by2 - by1 + 1.0)

        def body(scw, _):
            i = jnp.argmax(scw)
            xx1 = jnp.maximum(bx1[i], bx1)
            yy1 = jnp.maximum(by1[i], by1)
            xx2 = jnp.minimum(bx2[i], bx2)
            yy2 = jnp.minimum(by2[i], by2)
            w = jnp.maximum(0.0, xx2 - xx1 + 1.0)
            h = jnp.maximum(0.0, yy2 - yy1 + 1.0)
            inter = w * h
            iou = inter / (areas[i] + areas - inter)
            scw = jnp.where(iou > _NMS_THRESH, -jnp.inf, scw)
            scw = scw.at[i].set(-jnp.inf)
            return scw, i

        _, keep = lax.scan(body, top_s, None, length=_POST_NMS)
        return p[keep]

    kept = jax.vmap(nms_single)(proposals, sc)
    batch_ids = jnp.broadcast_to(
        jnp.arange(batch, dtype=jnp.float32)[:, None, None], (batch, _POST_NMS, 1))
    return jnp.concatenate([batch_ids, kept], axis=2)


# speedup vs baseline: 1.9245x; 1.9245x over previous
import numpy as np
import jax
import jax.numpy as jnp
from jax import lax
from jax.experimental import pallas as pl
from jax.experimental.pallas import tpu as pltpu

_FEAT_STRIDE = 16
_SCALES = np.array([8.0, 16.0, 32.0])
_RATIOS = np.array([0.5, 1.0, 2.0])
_PRE_NMS = 6000
_POST_NMS = 300
_NMS_THRESH = 0.7
_A = 9
_INT_MIN = np.int32(-(2 ** 31))


def _whctrs(a):
    w = a[2] - a[0] + 1.0
    h = a[3] - a[1] + 1.0
    xc = a[0] + 0.5 * (w - 1.0)
    yc = a[1] + 0.5 * (h - 1.0)
    return w, h, xc, yc


def _mkanchors(ws, hs, xc, yc):
    ws = ws[:, None]
    hs = hs[:, None]
    return np.hstack([xc - 0.5 * (ws - 1.0), yc - 0.5 * (hs - 1.0),
                      xc + 0.5 * (ws - 1.0), yc + 0.5 * (hs - 1.0)])


def _ratio_enum(a, ratios):
    w, h, xc, yc = _whctrs(a)
    size = w * h
    size_ratios = size / ratios
    ws = np.round(np.sqrt(size_ratios))
    hs = np.round(ws * ratios)
    return _mkanchors(ws, hs, xc, yc)


def _scale_enum(a, scales):
    w, h, xc, yc = _whctrs(a)
    ws = w * scales
    hs = h * scales
    return _mkanchors(ws, hs, xc, yc)


def _gen_anchors(base_size=16):
    base = np.array([0.0, 0.0, base_size - 1.0, base_size - 1.0])
    ra = _ratio_enum(base, _RATIOS)
    return np.vstack([_scale_enum(ra[i, :], _SCALES) for i in range(ra.shape[0])])


def _anchor_geom_amajor(fh, fw, npad):
    anc = _gen_anchors().astype(np.float32)
    sx = np.arange(fw, dtype=np.float32) * np.float32(_FEAT_STRIDE)
    sy = np.arange(fh, dtype=np.float32) * np.float32(_FEAT_STRIDE)
    SX, SY = np.meshgrid(sx, sy)
    shifts = np.stack([SX.ravel(), SY.ravel(), SX.ravel(), SY.ravel()], axis=1).astype(np.float32)
    a4 = (anc[:, None, :] + shifts[None, :, :]).reshape(_A * fh * fw, 4)
    w = (a4[:, 2] - a4[:, 0]) + np.float32(1.0)
    h = (a4[:, 3] - a4[:, 1]) + np.float32(1.0)
    cx = a4[:, 0] + np.float32(0.5) * w
    cy = a4[:, 1] + np.float32(0.5) * h
    hw = fh * fw
    ridx = (np.arange(_A * hw, dtype=np.int64) % hw) * _A + (np.arange(_A * hw, dtype=np.int64) // hw)
    ridx = ridx.astype(np.int32)
    n = _A * hw

    def pad(v, c):
        return np.concatenate([v, np.full((npad - n,), c, v.dtype)])

    return (pad(w, 1.0), pad(h, 1.0), pad(cx, 0.0), pad(cy, 0.0),
            pad(ridx, np.int32(10 ** 8)))


def _body(dx_r, dy_r, dw_r, dh_r, sc_r, aw_r, ah_r, acx_r, acy_r, ridx_r, imi_r,
          out_r, x1_r, y1_r, x2_r, y2_r, ar_r, ws_r):
    nb, n = sc_r.shape
    widths = aw_r[...]
    heights = ah_r[...]
    pcx = dx_r[...] * widths + acx_r[...]
    pcy = dy_r[...] * heights + acy_r[...]
    pw = jnp.exp(dw_r[...]) * widths
    ph = jnp.exp(dh_r[...]) * heights
    x1 = pcx - 0.5 * pw
    y1 = pcy - 0.5 * ph
    x2 = pcx + 0.5 * pw
    y2 = pcy + 0.5 * ph
    im_h = imi_r[:, 0:1]
    im_w = imi_r[:, 1:2]
    x1 = jnp.clip(x1, 0.0, im_w - 1.0)
    x2 = jnp.clip(x2, 0.0, im_w - 1.0)
    y1 = jnp.clip(y1, 0.0, im_h - 1.0)
    y2 = jnp.clip(y2, 0.0, im_h - 1.0)
    x1_r[...] = x1
    y1_r[...] = y1
    x2_r[...] = x2
    y2_r[...] = y2
    ar_r[...] = ((x2 - x1) + 1.0) * ((y2 - y1) + 1.0)

    sc = sc_r[...]
    bits = lax.bitcast_convert_type(sc, jnp.int32)
    key = jnp.where(bits >= 0, bits, _INT_MIN - bits)

    def bs_body(t, T):
        cand = T + (jnp.int32(1) << (jnp.int32(30) - t))
        cnt = jnp.sum((key >= cand).astype(jnp.int32), axis=1, keepdims=True)
        return jnp.where(cnt >= _PRE_NMS, cand, T)

    T = lax.fori_loop(0, 31, bs_body, jnp.full((nb, 1), _INT_MIN, jnp.int32))
    c_gt = jnp.sum((key > T).astype(jnp.int32), axis=1, keepdims=True)
    m = _PRE_NMS - c_gt
    ridx = ridx_r[...]
    tie = key == T

    def is_body(t, I):
        cand = I + (jnp.int32(1) << (jnp.int32(16) - t))
        f = jnp.sum((tie & (ridx < cand)).astype(jnp.int32), axis=1, keepdims=True)
        return jnp.where(f < m, cand, I)

    I = lax.fori_loop(0, 17, is_body, jnp.zeros((nb, 1), jnp.int32))
    part = (key > T) | (tie & (ridx <= I))
    ninf = jnp.float32(-jnp.inf)
    ws_r[...] = jnp.where(part, sc, ninf)

    lane = lax.broadcasted_iota(jnp.int32, (nb, 128), 1)
    z = jnp.zeros((nb, 1), jnp.float32)

    def step(i, fb):
        f1, f2, f3, f4 = fb
        ws = ws_r[...]
        mx = jnp.max(ws, axis=1, keepdims=True)
        selc = jnp.where(ws == mx, ridx, jnp.int32(10 ** 9))
        sel = jnp.min(selc, axis=1, keepdims=True)
        oh = ridx == sel
        bx1 = jnp.sum(jnp.where(oh, x1_r[...], 0.0), axis=1, keepdims=True)
        by1 = jnp.sum(jnp.where(oh, y1_r[...], 0.0), axis=1, keepdims=True)
        bx2 = jnp.sum(jnp.where(oh, x2_r[...], 0.0), axis=1, keepdims=True)
        by2 = jnp.sum(jnp.where(oh, y2_r[...], 0.0), axis=1, keepdims=True)
        bar = jnp.sum(jnp.where(oh, ar_r[...], 0.0), axis=1, keepdims=True)
        alive = mx > ninf
        ox1 = jnp.where(alive, bx1, f1)
        oy1 = jnp.where(alive, by1, f2)
        ox2 = jnp.where(alive, bx2, f3)
        oy2 = jnp.where(alive, by2, f4)
        isz = i == 0
        f1 = jnp.where(isz, bx1, f1)
        f2 = jnp.where(isz, by1, f2)
        f3 = jnp.where(isz, bx2, f3)
        f4 = jnp.where(isz, by2, f4)
        xx1 = jnp.maximum(bx1, x1_r[...])
        yy1 = jnp.maximum(by1, y1_r[...])
        xx2 = jnp.minimum(bx2, x2_r[...])
        yy2 = jnp.minimum(by2, y2_r[...])
        w = jnp.maximum(0.0, (xx2 - xx1) + 1.0)
        h = jnp.maximum(0.0, (yy2 - yy1) + 1.0)
        inter = w * h
        iou = inter / ((bar + ar_r[...]) - inter)
        ws_r[...] = jnp.where((iou > _NMS_THRESH) | oh, ninf, ws)
        tile = jnp.where(lane == 0, ox1,
                         jnp.where(lane == 1, oy1,
                                   jnp.where(lane == 2, ox2,
                                             jnp.where(lane == 3, oy2, 0.0))))
        out_r[i] = tile
        return (f1, f2, f3, f4)

    lax.fori_loop(0, _POST_NMS, step, (z, z, z, z))


def kernel(scores, bbox_deltas, im_info, cfg_key):
    batch = scores.shape[0]
    fh, fw = scores.shape[2], scores.shape[3]
    n = _A * fh * fw
    npad = ((n + 127) // 128) * 128
    padn = npad - n

    sc = scores[:, _A:, :, :].reshape(batch, n)
    dx = bbox_deltas[:, 0::4, :, :].reshape(batch, n)
    dy = bbox_deltas[:, 1::4, :, :].reshape(batch, n)
    dw = bbox_deltas[:, 2::4, :, :].reshape(batch, n)
    dh = bbox_deltas[:, 3::4, :, :].reshape(batch, n)
    sc = jnp.pad(sc, ((0, 0), (0, padn)), constant_values=-jnp.inf)
    dx, dy, dw, dh = (jnp.pad(v, ((0, 0), (0, padn))) for v in (dx, dy, dw, dh))
    aw, ah, acx, acy, ridx = _anchor_geom_amajor(fh, fw, npad)
    aw, ah, acx, acy, ridx = (jnp.asarray(v)[None, :] for v in (aw, ah, acx, acy, ridx))
    imi = jnp.pad(im_info, ((0, 0), (0, 126)))

    out = pl.pallas_call(
        _body,
        out_shape=jax.ShapeDtypeStruct((_POST_NMS, batch, 128), jnp.float32),
        scratch_shapes=[pltpu.VMEM((batch, npad), jnp.float32)] * 6,
    )(dx, dy, dw, dh, sc, aw, ah, acx, acy, ridx, imi)

    kept = out[:, :, 0:4].transpose(1, 0, 2)
    batch_ids = jnp.broadcast_to(
        jnp.arange(batch, dtype=jnp.float32)[:, None, None], (batch, _POST_NMS, 1))
    return jnp.concatenate([batch_ids, kept], axis=2)


# speedup vs baseline: 2.1165x; 1.0998x over previous
import numpy as np
import jax
import jax.numpy as jnp
from jax import lax
from jax.experimental import pallas as pl
from jax.experimental.pallas import tpu as pltpu

_FEAT_STRIDE = 16
_SCALES = np.array([8.0, 16.0, 32.0])
_RATIOS = np.array([0.5, 1.0, 2.0])
_PRE_NMS = 6000
_POST_NMS = 300
_NMS_THRESH = 0.7
_A = 9
_INT_MIN = np.int32(-(2 ** 31))


def _whctrs(a):
    w = a[2] - a[0] + 1.0
    h = a[3] - a[1] + 1.0
    xc = a[0] + 0.5 * (w - 1.0)
    yc = a[1] + 0.5 * (h - 1.0)
    return w, h, xc, yc


def _mkanchors(ws, hs, xc, yc):
    ws = ws[:, None]
    hs = hs[:, None]
    return np.hstack([xc - 0.5 * (ws - 1.0), yc - 0.5 * (hs - 1.0),
                      xc + 0.5 * (ws - 1.0), yc + 0.5 * (hs - 1.0)])


def _ratio_enum(a, ratios):
    w, h, xc, yc = _whctrs(a)
    size = w * h
    size_ratios = size / ratios
    ws = np.round(np.sqrt(size_ratios))
    hs = np.round(ws * ratios)
    return _mkanchors(ws, hs, xc, yc)


def _scale_enum(a, scales):
    w, h, xc, yc = _whctrs(a)
    ws = w * scales
    hs = h * scales
    return _mkanchors(ws, hs, xc, yc)


def _gen_anchors(base_size=16):
    base = np.array([0.0, 0.0, base_size - 1.0, base_size - 1.0])
    ra = _ratio_enum(base, _RATIOS)
    return np.vstack([_scale_enum(ra[i, :], _SCALES) for i in range(ra.shape[0])])


def _anchor_geom_amajor(fh, fw, npad):
    anc = _gen_anchors().astype(np.float32)
    sx = np.arange(fw, dtype=np.float32) * np.float32(_FEAT_STRIDE)
    sy = np.arange(fh, dtype=np.float32) * np.float32(_FEAT_STRIDE)
    SX, SY = np.meshgrid(sx, sy)
    shifts = np.stack([SX.ravel(), SY.ravel(), SX.ravel(), SY.ravel()], axis=1).astype(np.float32)
    a4 = (anc[:, None, :] + shifts[None, :, :]).reshape(_A * fh * fw, 4)
    w = (a4[:, 2] - a4[:, 0]) + np.float32(1.0)
    h = (a4[:, 3] - a4[:, 1]) + np.float32(1.0)
    cx = a4[:, 0] + np.float32(0.5) * w
    cy = a4[:, 1] + np.float32(0.5) * h
    hw = fh * fw
    ridx = (np.arange(_A * hw, dtype=np.int64) % hw) * _A + (np.arange(_A * hw, dtype=np.int64) // hw)
    ridx = ridx.astype(np.int32)
    n = _A * hw

    def pad(v, c):
        return np.concatenate([v, np.full((npad - n,), c, v.dtype)])

    return (pad(w, 1.0), pad(h, 1.0), pad(cx, 0.0), pad(cy, 0.0),
            pad(ridx, np.int32(10 ** 8)))


def _body(dx_r, dy_r, dw_r, dh_r, sc_r, aw_r, ah_r, acx_r, acy_r, ridx_r, imi_r,
          out_r, x1_r, y1_r, x2_r, y2_r, ar_r, ws_r):
    nb, n = sc_r.shape
    widths = aw_r[...]
    heights = ah_r[...]
    pcx = dx_r[...] * widths + acx_r[...]
    pcy = dy_r[...] * heights + acy_r[...]
    pw = jnp.exp(dw_r[...]) * widths
    ph = jnp.exp(dh_r[...]) * heights
    x1 = pcx - 0.5 * pw
    y1 = pcy - 0.5 * ph
    x2 = pcx + 0.5 * pw
    y2 = pcy + 0.5 * ph
    im_h = imi_r[:, 0:1]
    im_w = imi_r[:, 1:2]
    x1 = jnp.clip(x1, 0.0, im_w - 1.0)
    x2 = jnp.clip(x2, 0.0, im_w - 1.0)
    y1 = jnp.clip(y1, 0.0, im_h - 1.0)
    y2 = jnp.clip(y2, 0.0, im_h - 1.0)
    x1_r[...] = x1
    y1_r[...] = y1
    x2_r[...] = x2
    y2_r[...] = y2
    ar_r[...] = ((x2 - x1) + 1.0) * ((y2 - y1) + 1.0)

    sc = sc_r[...]
    bits = lax.bitcast_convert_type(sc, jnp.int32)
    key = jnp.where(bits >= 0, bits, _INT_MIN - bits)

    def bs_body(t, T):
        cand = T + (jnp.int32(1) << (jnp.int32(30) - t))
        cnt = jnp.sum((key >= cand).astype(jnp.int32), axis=1, keepdims=True)
        return jnp.where(cnt >= _PRE_NMS, cand, T)

    T = lax.fori_loop(0, 31, bs_body, jnp.full((nb, 1), _INT_MIN, jnp.int32))
    c_gt = jnp.sum((key > T).astype(jnp.int32), axis=1, keepdims=True)
    m = _PRE_NMS - c_gt
    ridx = ridx_r[...]
    tie = key == T

    def is_body(t, I):
        cand = I + (jnp.int32(1) << (jnp.int32(16) - t))
        f = jnp.sum((tie & (ridx < cand)).astype(jnp.int32), axis=1, keepdims=True)
        return jnp.where(f < m, cand, I)

    I = lax.fori_loop(0, 17, is_body, jnp.zeros((nb, 1), jnp.int32))
    part = (key > T) | (tie & (ridx <= I))
    ninf = jnp.float32(-jnp.inf)
    ws_r[...] = jnp.where(part, sc, ninf)

    lane = lax.broadcasted_iota(jnp.int32, (nb, 128), 1)
    z = jnp.zeros((nb, 1), jnp.float32)

    def step(i, fb):
        f1, f2, f3, f4 = fb
        ws = ws_r[...]
        mx = jnp.max(ws, axis=1, keepdims=True)
        selc = jnp.where(ws == mx, ridx, jnp.int32(10 ** 9))
        sel = jnp.min(selc, axis=1, keepdims=True)
        oh = ridx == sel
        bx1 = jnp.sum(jnp.where(oh, x1_r[...], 0.0), axis=1, keepdims=True)
        by1 = jnp.sum(jnp.where(oh, y1_r[...], 0.0), axis=1, keepdims=True)
        bx2 = jnp.sum(jnp.where(oh, x2_r[...], 0.0), axis=1, keepdims=True)
        by2 = jnp.sum(jnp.where(oh, y2_r[...], 0.0), axis=1, keepdims=True)
        bar = ((bx2 - bx1) + 1.0) * ((by2 - by1) + 1.0)
        alive = mx > ninf
        ox1 = jnp.where(alive, bx1, f1)
        oy1 = jnp.where(alive, by1, f2)
        ox2 = jnp.where(alive, bx2, f3)
        oy2 = jnp.where(alive, by2, f4)
        isz = i == 0
        f1 = jnp.where(isz, bx1, f1)
        f2 = jnp.where(isz, by1, f2)
        f3 = jnp.where(isz, bx2, f3)
        f4 = jnp.where(isz, by2, f4)
        xx1 = jnp.maximum(bx1, x1_r[...])
        yy1 = jnp.maximum(by1, y1_r[...])
        xx2 = jnp.minimum(bx2, x2_r[...])
        yy2 = jnp.minimum(by2, y2_r[...])
        w = jnp.maximum(0.0, (xx2 - xx1) + 1.0)
        h = jnp.maximum(0.0, (yy2 - yy1) + 1.0)
        inter = w * h
        iou = inter / ((bar + ar_r[...]) - inter)
        ws_r[...] = jnp.where(iou > _NMS_THRESH, ninf, ws)
        tile = jnp.where(lane == 0, ox1,
                         jnp.where(lane == 1, oy1,
                                   jnp.where(lane == 2, ox2,
                                             jnp.where(lane == 3, oy2, 0.0))))
        out_r[i] = tile
        return (f1, f2, f3, f4)

    lax.fori_loop(0, _POST_NMS, step, (z, z, z, z))


def kernel(scores, bbox_deltas, im_info, cfg_key):
    batch = scores.shape[0]
    fh, fw = scores.shape[2], scores.shape[3]
    n = _A * fh * fw
    npad = ((n + 127) // 128) * 128
    padn = npad - n

    sc = scores[:, _A:, :, :].reshape(batch, n)
    dx = bbox_deltas[:, 0::4, :, :].reshape(batch, n)
    dy = bbox_deltas[:, 1::4, :, :].reshape(batch, n)
    dw = bbox_deltas[:, 2::4, :, :].reshape(batch, n)
    dh = bbox_deltas[:, 3::4, :, :].reshape(batch, n)
    sc = jnp.pad(sc, ((0, 0), (0, padn)), constant_values=-jnp.inf)
    dx, dy, dw, dh = (jnp.pad(v, ((0, 0), (0, padn))) for v in (dx, dy, dw, dh))
    aw, ah, acx, acy, ridx = _anchor_geom_amajor(fh, fw, npad)
    aw, ah, acx, acy, ridx = (jnp.asarray(v)[None, :] for v in (aw, ah, acx, acy, ridx))
    imi = jnp.pad(im_info, ((0, 0), (0, 126)))

    out = pl.pallas_call(
        _body,
        out_shape=jax.ShapeDtypeStruct((_POST_NMS, batch, 128), jnp.float32),
        scratch_shapes=[pltpu.VMEM((batch, npad), jnp.float32)] * 6,
    )(dx, dy, dw, dh, sc, aw, ah, acx, acy, ridx, imi)

    kept = out[:, :, 0:4].transpose(1, 0, 2)
    batch_ids = jnp.broadcast_to(
        jnp.arange(batch, dtype=jnp.float32)[:, None, None], (batch, _POST_NMS, 1))
    return jnp.concatenate([batch_ids, kept], axis=2)
